# baseline (speedup 1.0000x reference)
import functools

import jax
import jax.numpy as jnp
from jax import lax
from jax.experimental import pallas as pl
from jax.experimental.pallas import tpu as pltpu
from jax.experimental.pallas import tpu_sc as plsc

CORPUS = 1_000_000
D = 64
NI, NJ = 16384, 50
B = NI * NJ
NW = 32
NIT = NI // 128
NBLK = NJ * NIT
BPW = NBLK // NW

_mesh = plsc.VectorSubcoreMesh(core_axis_name="c", subcore_axis_name="s")


@functools.partial(
    pl.kernel,
    out_type=jax.ShapeDtypeStruct((NJ, 8, 128, 8, 128), jnp.float32),
    mesh=_mesh,
    scratch_types=[
        pltpu.VMEM((2, 128), jnp.int32),
        pltpu.VMEM((2, 128, D), jnp.float32),
        pltpu.VMEM((2, D, 128), jnp.float32),
        pltpu.SemaphoreType.DMA,
        pltpu.SemaphoreType.DMA,
        pltpu.SemaphoreType.DMA,
        pltpu.SemaphoreType.DMA,
        pltpu.SemaphoreType.DMA,
        pltpu.SemaphoreType.DMA,
    ],
    compiler_params=pltpu.CompilerParams(
        use_tc_tiling_on_sc=False, needs_layout_passes=False
    ),
)
def _emb_lookup(idx_hbm, table_hbm, out_hbm, idx_v, rows_v, obuf_v,
                gi0, gi1, gr0, gr1, po0, po1):
    wid = lax.axis_index("s") * 2 + lax.axis_index("c")
    isem = (gi0, gi1)
    gsem = (gr0, gr1)
    osem = (po0, po1)

    def block_id(t):
        return wid * BPW + t

    def fetch(t, s):
        b = block_id(t)
        n0 = (b // NIT) * NI + (b % NIT) * 128
        pltpu.async_copy(idx_hbm.at[pl.ds(n0, 128)], idx_v.at[s], isem[s])

    def fetch_wait(t, s):
        b = block_id(t)
        n0 = (b // NIT) * NI + (b % NIT) * 128
        pltpu.make_async_copy(
            idx_hbm.at[pl.ds(n0, 128)], idx_v.at[s], isem[s]
        ).wait()
        pltpu.async_copy(table_hbm.at[idx_v.at[s]], rows_v.at[s], gsem[s])

    def gather_wait(t, s):
        pltpu.make_async_copy(
            table_hbm.at[idx_v.at[s]], rows_v.at[s], gsem[s]
        ).wait()

    def put(t, s):
        b = block_id(t)
        for kt in range(8):
            pltpu.async_copy(
                obuf_v.at[s, pl.ds(kt * 8, 8)],
                out_hbm.at[b // NIT, kt, b % NIT],
                osem[s],
            )

    def put_wait(t, s):
        b = block_id(t)
        for kt in range(8):
            pltpu.make_async_copy(
                obuf_v.at[s, pl.ds(kt * 8, 8)],
                out_hbm.at[b // NIT, kt, b % NIT],
                osem[s],
            ).wait()

    def transpose(s):
        rows = rows_v.at[s]
        obuf = obuf_v.at[s]
        lane = lax.iota(jnp.int32, 16)

        for q in range(8):
            rb = lane + (16 * q)

            def kbody(k, rb=rb, q=q):
                col = jax.lax.broadcast(k, (16,))
                obuf[k, pl.ds(16 * q, 16)] = plsc.load_gather(rows, [rb, col])

            plsc.parallel_loop(0, D, 1, unroll=16)(kbody)

    fetch(0, 0)
    fetch(1, 1)
    fetch_wait(0, 0)

    def body(p, _):
        a = 2 * p
        fetch_wait(a + 1, 1)

        gather_wait(a, 0)
        @pl.when(p >= 1)
        def _():
            put_wait(a - 2, 0)
        transpose(0)
        @pl.when(a + 2 < BPW)
        def _():
            fetch(a + 2, 0)
        put(a, 0)
        @pl.when(a + 2 < BPW)
        def _():
            fetch_wait(a + 2, 0)

        gather_wait(a + 1, 1)
        @pl.when(p >= 1)
        def _():
            put_wait(a - 1, 1)
        transpose(1)
        @pl.when(a + 3 < BPW)
        def _():
            fetch(a + 3, 1)
        put(a + 1, 1)
        return 0

    lax.fori_loop(0, BPW // 2, body, 0)
    put_wait(BPW - 2, 0)
    put_wait(BPW - 1, 1)


def kernel(idx, table):
    idx_t = jnp.transpose(idx).reshape(-1).astype(jnp.int32)
    out_v = _emb_lookup(idx_t, table)
    return out_v.transpose(2, 4, 0, 1, 3).reshape(NI, NJ, D)


# speedup vs baseline: 1.0024x; 1.0024x over previous
import functools

import jax
import jax.numpy as jnp
from jax import lax
from jax.experimental import pallas as pl
from jax.experimental.pallas import tpu as pltpu
from jax.experimental.pallas import tpu_sc as plsc

CORPUS = 1_000_000
D = 64
NI, NJ = 16384, 50
B = NI * NJ
NW = 32
NIT = NI // 128
NBLK = NJ * NIT
BPW = NBLK // NW

_mesh = plsc.VectorSubcoreMesh(core_axis_name="c", subcore_axis_name="s")


@functools.partial(
    pl.kernel,
    out_type=jax.ShapeDtypeStruct((NJ, 8, 128, 8, 128), jnp.float32),
    mesh=_mesh,
    scratch_types=[
        pltpu.VMEM((2, 128), jnp.int32),
        pltpu.VMEM((2, 128, D), jnp.float32),
        pltpu.VMEM((2, D, 128), jnp.float32),
        pltpu.SemaphoreType.DMA,
        pltpu.SemaphoreType.DMA,
        pltpu.SemaphoreType.DMA,
        pltpu.SemaphoreType.DMA,
        pltpu.SemaphoreType.DMA,
        pltpu.SemaphoreType.DMA,
    ],
    compiler_params=pltpu.CompilerParams(
        use_tc_tiling_on_sc=False, needs_layout_passes=False
    ),
)
def _emb_lookup(idx_hbm, table_hbm, out_hbm, idx_v, rows_v, obuf_v,
                gi0, gi1, gr0, gr1, po0, po1):
    wid = lax.axis_index("s") * 2 + lax.axis_index("c")
    isem = (gi0, gi1)
    gsem = (gr0, gr1)
    osem = (po0, po1)

    def block_id(t):
        return wid * BPW + t

    def fetch(t, s):
        b = block_id(t)
        n0 = (b // NIT) * NI + (b % NIT) * 128
        pltpu.async_copy(idx_hbm.at[pl.ds(n0, 128)], idx_v.at[s], isem[s])

    def fetch_wait(t, s):
        b = block_id(t)
        n0 = (b // NIT) * NI + (b % NIT) * 128
        pltpu.make_async_copy(
            idx_hbm.at[pl.ds(n0, 128)], idx_v.at[s], isem[s]
        ).wait()
        pltpu.async_copy(table_hbm.at[idx_v.at[s]], rows_v.at[s], gsem[s])

    def gather_wait(t, s):
        pltpu.make_async_copy(
            table_hbm.at[idx_v.at[s]], rows_v.at[s], gsem[s]
        ).wait()

    def put(t, s):
        b = block_id(t)
        for kt in range(8):
            pltpu.async_copy(
                obuf_v.at[s, pl.ds(kt * 8, 8)],
                out_hbm.at[b // NIT, kt, b % NIT],
                osem[s],
            )

    def put_wait(t, s):
        b = block_id(t)
        for kt in range(8):
            pltpu.make_async_copy(
                obuf_v.at[s, pl.ds(kt * 8, 8)],
                out_hbm.at[b // NIT, kt, b % NIT],
                osem[s],
            ).wait()

    def transpose(s):
        rows = rows_v.at[s]
        obuf = obuf_v.at[s]
        lane = lax.iota(jnp.int32, 16)

        for q in range(8):
            rb = lane + (16 * q)

            def kbody(k, rb=rb, q=q):
                kv = jax.lax.broadcast(k, (16,))
                rvec = plsc.load_gather(rows, [rb, kv])
                plsc.store_scatter(obuf, [kv, lane + 16 * q], rvec)

            plsc.parallel_loop(0, D, 1, unroll=16)(kbody)

    fetch(0, 0)
    fetch(1, 1)
    fetch_wait(0, 0)

    def body(p, _):
        a = 2 * p
        fetch_wait(a + 1, 1)

        gather_wait(a, 0)
        @pl.when(p >= 1)
        def _():
            put_wait(a - 2, 0)
        transpose(0)
        @pl.when(a + 2 < BPW)
        def _():
            fetch(a + 2, 0)
        put(a, 0)
        @pl.when(a + 2 < BPW)
        def _():
            fetch_wait(a + 2, 0)

        gather_wait(a + 1, 1)
        @pl.when(p >= 1)
        def _():
            put_wait(a - 1, 1)
        transpose(1)
        @pl.when(a + 3 < BPW)
        def _():
            fetch(a + 3, 1)
        put(a + 1, 1)
        return 0

    lax.fori_loop(0, BPW // 2, body, 0)
    put_wait(BPW - 2, 0)
    put_wait(BPW - 1, 1)


def kernel(idx, table):
    idx_t = jnp.transpose(idx).reshape(-1).astype(jnp.int32)
    out_v = _emb_lookup(idx_t, table)
    return out_v.transpose(2, 4, 0, 1, 3).reshape(NI, NJ, D)


# speedup vs baseline: 1.1017x; 1.0991x over previous
import functools

import jax
import jax.numpy as jnp
from jax import lax
from jax.experimental import pallas as pl
from jax.experimental.pallas import tpu as pltpu
from jax.experimental.pallas import tpu_sc as plsc

CORPUS = 1_000_000
D = 64
B = 16384 * 50
NW = 32
BPW = B // NW
CHUNK = 640
NCHUNK = BPW // CHUNK
NPAIR = NCHUNK // 2

_mesh = plsc.VectorSubcoreMesh(core_axis_name="c", subcore_axis_name="s")


@functools.partial(
    pl.kernel,
    out_type=jax.ShapeDtypeStruct((B, D), jnp.float32),
    mesh=_mesh,
    scratch_types=[
        pltpu.VMEM((BPW,), jnp.int32),
        pltpu.VMEM((CHUNK, D), jnp.float32),
        pltpu.VMEM((CHUNK, D), jnp.float32),
        pltpu.SemaphoreType.DMA,
        pltpu.SemaphoreType.DMA,
        pltpu.SemaphoreType.DMA,
        pltpu.SemaphoreType.DMA,
    ],
    compiler_params=pltpu.CompilerParams(use_tc_tiling_on_sc=False),
)
def _emb_lookup(idx_hbm, table_hbm, out_hbm, idx_v, buf0, buf1, g0, g1, o0, o1):
    wid = lax.axis_index("s") * 2 + lax.axis_index("c")
    base = wid * BPW
    pltpu.sync_copy(idx_hbm.at[pl.ds(base, BPW)], idx_v)

    def gather(c, buf, sem):
        pltpu.async_copy(table_hbm.at[idx_v.at[pl.ds(c * CHUNK, CHUNK)]], buf, sem)

    def gather_wait(c, buf, sem):
        pltpu.make_async_copy(
            table_hbm.at[idx_v.at[pl.ds(c * CHUNK, CHUNK)]], buf, sem
        ).wait()

    def put(c, buf, sem):
        pltpu.async_copy(buf, out_hbm.at[pl.ds(base + c * CHUNK, CHUNK)], sem)

    def put_wait(c, buf, sem):
        pltpu.make_async_copy(
            buf, out_hbm.at[pl.ds(base + c * CHUNK, CHUNK)], sem
        ).wait()

    gather(0, buf0, g0)
    gather(1, buf1, g1)

    def body(p, _):
        a = 2 * p
        gather_wait(a, buf0, g0)
        put(a, buf0, o0)
        gather_wait(a + 1, buf1, g1)
        put(a + 1, buf1, o1)

        @pl.when(p < NPAIR - 1)
        def _():
            put_wait(a, buf0, o0)
            gather(a + 2, buf0, g0)
            put_wait(a + 1, buf1, o1)
            gather(a + 3, buf1, g1)

        return 0

    lax.fori_loop(0, NPAIR, body, 0)
    put_wait(NCHUNK - 2, buf0, o0)
    put_wait(NCHUNK - 1, buf1, o1)


def kernel(idx, table):
    idx_flat = idx.reshape(-1).astype(jnp.int32)
    out = _emb_lookup(idx_flat, table)
    return out.reshape(idx.shape + (D,))
